# baseline (speedup 1.0000x reference)
import functools

import jax
import jax.numpy as jnp
from jax import lax
from jax.experimental import pallas as pl
from jax.experimental.pallas import tpu as pltpu
from jax.experimental.pallas import tpu_sc as plsc

B = 1024
L = 20
D = 128
V = 1000

N_IDX = B * L

_INFO = plsc.get_sparse_core_info()
NC = _INFO.num_cores
NW = NC * _INFO.num_subcores
ROWS_PER_W = N_IDX // NW
CHUNK = 128
N_CHUNKS = ROWS_PER_W // CHUNK
B_CHUNKS = B // CHUNK


def _gather_body(emb_hbm, idx_hbm, out_hbm, idx_v, rows_v, emb_sp, isem, sem):
    sid = lax.axis_index("s")
    wid = sid * NC + lax.axis_index("c")
    base = wid * ROWS_PER_W
    @pl.when(sid < 5)
    def _stage():
        pltpu.sync_copy(
            emb_hbm.at[pl.ds(sid * 200, 200)], emb_sp.at[pl.ds(sid * 200, 200)]
        )
    plsc.subcore_barrier()
    idx_copies = []
    for k in range(N_CHUNKS):
        t = wid * N_CHUNKS + k
        l_row = t // B_CHUNKS
        col = (t % B_CHUNKS) * CHUNK
        idx_copies.append(
            pltpu.async_copy(
                idx_hbm.at[l_row, pl.ds(col, CHUNK)],
                idx_v.at[pl.ds(k * CHUNK, CHUNK)],
                isem,
            )
        )
    for c in idx_copies:
        c.wait()
    copies = []
    for j in range(N_CHUNKS):
        copies.append(
            pltpu.async_copy(
                emb_sp.at[idx_v.at[pl.ds(j * CHUNK, CHUNK)]],
                rows_v.at[pl.ds(j * CHUNK, CHUNK)],
                sem,
            )
        )
    writes = []
    for j in range(N_CHUNKS):
        copies[j].wait()
        writes.append(
            pltpu.async_copy(
                rows_v.at[pl.ds(j * CHUNK, CHUNK)],
                out_hbm.at[pl.ds(base + j * CHUNK, CHUNK)],
                isem,
            )
        )
    for c in writes:
        c.wait()


@functools.partial(
    pl.kernel,
    out_type=jax.ShapeDtypeStruct((N_IDX, D), jnp.float32),
    mesh=plsc.VectorSubcoreMesh(core_axis_name="c", subcore_axis_name="s"),
    scratch_types=[
        pltpu.VMEM((ROWS_PER_W,), jnp.int32),
        pltpu.VMEM((ROWS_PER_W, D), jnp.float32),
        pltpu.VMEM_SHARED((V, D), jnp.float32),
        pltpu.SemaphoreType.DMA,
        pltpu.SemaphoreType.DMA,
    ],
)
def _sc_gather(emb_hbm, idx_hbm, out_hbm, idx_v, rows_v, emb_sp, isem, sem):
    _gather_body(emb_hbm, idx_hbm, out_hbm, idx_v, rows_v, emb_sp, isem, sem)


BS = 512


def _matmul_body(x_ref, w_ref, b_ref, out_ref):
    pieces = [x_ref[l].astype(jnp.bfloat16) for l in range(L)]
    x = jnp.concatenate(pieces, axis=1)
    out_ref[:] = (
        lax.dot_general(
            w_ref[:],
            x,
            dimension_numbers=(((1,), (1,)), ((), ())),
            preferred_element_type=jnp.float32,
        )
        + b_ref[:]
    )


def _tc_matmul_t(x3, W, bcol):
    return pl.pallas_call(
        _matmul_body,
        grid=(B // BS,),
        in_specs=[
            pl.BlockSpec((L, BS, D), lambda i: (0, i, 0)),
            pl.BlockSpec((V, L * D), lambda i: (0, 0)),
            pl.BlockSpec((V, 1), lambda i: (0, 0)),
        ],
        out_specs=pl.BlockSpec((V, BS), lambda i: (0, i)),
        out_shape=jax.ShapeDtypeStruct((V, B), jnp.float32),
    )(x3, W, bcol)


def kernel(idx, emb, W, b):
    idx_lm = idx.T
    W16 = W.astype(jnp.bfloat16)
    x = _sc_gather(emb, idx_lm)
    x3 = x.reshape(L, B, D)
    out_t = _tc_matmul_t(x3, W16, b.reshape(V, 1))
    return out_t.T


# speedup vs baseline: 1.0353x; 1.0353x over previous
import functools

import jax
import jax.numpy as jnp
from jax import lax
from jax.experimental import pallas as pl
from jax.experimental.pallas import tpu as pltpu
from jax.experimental.pallas import tpu_sc as plsc

B = 1024
L = 20
D = 128
V = 1000

N_IDX = B * L

_INFO = plsc.get_sparse_core_info()
NC = _INFO.num_cores
NW = NC * _INFO.num_subcores
ROWS_PER_W = N_IDX // NW
CHUNK = 128
N_CHUNKS = ROWS_PER_W // CHUNK
B_CHUNKS = B // CHUNK


def _gather_body(emb_hbm, idx_hbm, out_hbm, idx_v, rows_v, emb_sp, isem, sem):
    sid = lax.axis_index("s")
    wid = sid * NC + lax.axis_index("c")
    base = wid * ROWS_PER_W
    idx_copies = []
    for k in range(N_CHUNKS):
        t = wid * N_CHUNKS + k
        l_row = t // B_CHUNKS
        col = (t % B_CHUNKS) * CHUNK
        idx_copies.append(
            pltpu.async_copy(
                idx_hbm.at[l_row, pl.ds(col, CHUNK)],
                idx_v.at[pl.ds(k * CHUNK, CHUNK)],
                isem,
            )
        )
    @pl.when(sid < 5)
    def _stage():
        pltpu.sync_copy(
            emb_hbm.at[pl.ds(sid * 200, 200)], emb_sp.at[pl.ds(sid * 200, 200)]
        )
    plsc.subcore_barrier()
    copies = []
    for j in range(N_CHUNKS):
        idx_copies[j].wait()
        copies.append(
            pltpu.async_copy(
                emb_sp.at[idx_v.at[pl.ds(j * CHUNK, CHUNK)]],
                rows_v.at[pl.ds(j * CHUNK, CHUNK)],
                sem,
            )
        )
    writes = []
    for j in range(N_CHUNKS):
        copies[j].wait()
        writes.append(
            pltpu.async_copy(
                rows_v.at[pl.ds(j * CHUNK, CHUNK)],
                out_hbm.at[pl.ds(base + j * CHUNK, CHUNK)],
                isem,
            )
        )
    for c in writes:
        c.wait()


@functools.partial(
    pl.kernel,
    out_type=jax.ShapeDtypeStruct((N_IDX, D), jnp.float32),
    mesh=plsc.VectorSubcoreMesh(core_axis_name="c", subcore_axis_name="s"),
    scratch_types=[
        pltpu.VMEM((ROWS_PER_W,), jnp.int32),
        pltpu.VMEM((ROWS_PER_W, D), jnp.float32),
        pltpu.VMEM_SHARED((V, D), jnp.float32),
        pltpu.SemaphoreType.DMA,
        pltpu.SemaphoreType.DMA,
    ],
)
def _sc_gather(emb_hbm, idx_hbm, out_hbm, idx_v, rows_v, emb_sp, isem, sem):
    _gather_body(emb_hbm, idx_hbm, out_hbm, idx_v, rows_v, emb_sp, isem, sem)


BS = 512


def _matmul_body(x_ref, w_ref, b_ref, out_ref):
    x = jnp.concatenate([x_ref[l] for l in range(L)], axis=1)
    out_ref[:] = (
        lax.dot_general(
            w_ref[:],
            x,
            dimension_numbers=(((1,), (1,)), ((), ())),
            preferred_element_type=jnp.float32,
        )
        + b_ref[:]
    )


def _tc_matmul_t(x3, W, bcol):
    return pl.pallas_call(
        _matmul_body,
        grid=(B // BS,),
        in_specs=[
            pl.BlockSpec((L, BS, D), lambda i: (0, i, 0)),
            pl.BlockSpec((V, L * D), lambda i: (0, 0)),
            pl.BlockSpec((V, 1), lambda i: (0, 0)),
        ],
        out_specs=pl.BlockSpec((V, BS), lambda i: (0, i)),
        out_shape=jax.ShapeDtypeStruct((V, B), jnp.float32),
    )(x3, W, bcol)


def kernel(idx, emb, W, b):
    idx_lm = idx.T
    x = _sc_gather(emb, idx_lm)
    x3 = x.reshape(L, B, D)
    out_t = _tc_matmul_t(x3, W, b.reshape(V, 1))
    return out_t.T
